# baseline (speedup 1.0000x reference)
import functools

import jax
import jax.numpy as jnp
from jax import lax
from jax.experimental import pallas as pl
from jax.experimental.pallas import tpu as pltpu
from jax.experimental.pallas import tpu_sc as plsc

NUM_CLASSES = 1000000
DIM = 64
BATCH = 1024
SEQ = 200
BL = BATCH * SEQ

NUM_WORKERS = 32
ROWS_PER_WORKER = BL // NUM_WORKERS
CHUNK = 128
NUM_CHUNKS = ROWS_PER_WORKER // CHUNK
NUM_PAIRS = NUM_CHUNKS // 2

_INV2PI = 0.15915494309189535
_MAGIC = 1.5 * 2.0**23
_C1 = 6.28125
_C2 = 6.283185307179586 - 6.28125

_S0, _S1, _S2, _S3 = (9.9945015e-01, -1.6583844e-01, 7.9985755e-03,
                      -1.4774044e-04)
_D0, _D1, _D2, _D3, _D4 = (9.9997109e-01, -4.9983761e-01, 4.1522305e-02,
                           -1.3441069e-03, 1.9065215e-05)


def _sincos(p):
    k = (p * _INV2PI + _MAGIC) - _MAGIC
    r = (p - k * _C1) - k * _C2
    t = r * r
    t2 = t * t
    s = ((_S0 + _S1 * t) + t2 * (_S2 + _S3 * t)) * r
    c_ = (_D0 + _D1 * t) + t2 * ((_D2 + _D3 * t) + t2 * _D4)
    return s, c_


def _body(words_hbm, tab_hbm, out_hbm,
          idx0, idx1, buf0, buf1, ob0, ob1,
          gsem0, gsem1, wsem0, wsem1):
    nc = 2
    wid = lax.axis_index("s") * nc + lax.axis_index("c")
    chunk0 = wid * NUM_CHUNKS

    def compute(buf, ob, l):
        posv = jnp.full((16,), (l + 1).astype(jnp.float32), jnp.float32)

        @plsc.parallel_loop(0, CHUNK, unroll=8)
        def row_body(r):
            for j in range(DIM // 16):
                a = buf[r, pl.ds(16 * j, 16)]
                f = buf[r, pl.ds(DIM + 16 * j, 16)]
                s, c = _sincos(f * posv)
                ob[r, pl.ds(16 * j, 16)] = a * c
                ob[r, pl.ds(DIM + 16 * j, 16)] = a * s

    def start_gather(g, idx, buf, gsem):
        pltpu.sync_copy(words_hbm.at[pl.ds(g * CHUNK, CHUNK)], idx)
        pltpu.make_async_copy(tab_hbm.at[idx], buf, gsem).start()

    def wait_gather(idx, buf, gsem):
        pltpu.make_async_copy(tab_hbm.at[idx], buf, gsem).wait()

    def out_block(g):
        q0 = g * CHUNK
        l = q0 // BATCH
        b0 = q0 % BATCH
        return out_hbm.at[pl.ds(b0, CHUNK), l], l

    start_gather(chunk0, idx0, buf0, gsem0)

    def pair_body(i, carry):
        g0 = chunk0 + 2 * i
        g1 = g0 + 1
        dst0, l0 = out_block(g0)
        dst1, l1 = out_block(g1)

        start_gather(g1, idx1, buf1, gsem1)

        wait_gather(idx0, buf0, gsem0)

        @pl.when(i > 0)
        def _():
            pltpu.make_async_copy(ob0, dst0, wsem0).wait()

        compute(buf0, ob0, l0)
        pltpu.make_async_copy(ob0, dst0, wsem0).start()

        @pl.when(i < NUM_PAIRS - 1)
        def _():
            start_gather(g0 + 2, idx0, buf0, gsem0)

        wait_gather(idx1, buf1, gsem1)

        @pl.when(i > 0)
        def _():
            pltpu.make_async_copy(ob1, dst1, wsem1).wait()

        compute(buf1, ob1, l1)
        pltpu.make_async_copy(ob1, dst1, wsem1).start()
        return carry

    lax.fori_loop(0, NUM_PAIRS, pair_body, 0)

    dstl0, _ = out_block(chunk0 + NUM_CHUNKS - 2)
    dstl1, _ = out_block(chunk0 + NUM_CHUNKS - 1)
    pltpu.make_async_copy(ob0, dstl0, wsem0).wait()
    pltpu.make_async_copy(ob1, dstl1, wsem1).wait()


_TBLOCK = 512
_TGRID = -(-NUM_CLASSES // _TBLOCK)


def _concat_body(a_ref, f_ref, o_ref):
    o_ref[:, :DIM] = a_ref[...]
    o_ref[:, DIM:] = f_ref[...]


def _build_table(amp_row, freq_row):
    return pl.pallas_call(
        _concat_body,
        grid=(_TGRID,),
        in_specs=[
            pl.BlockSpec((_TBLOCK, DIM), lambda g: (g, 0)),
            pl.BlockSpec((_TBLOCK, DIM), lambda g: (g, 0)),
        ],
        out_specs=pl.BlockSpec((_TBLOCK, 2 * DIM), lambda g: (g, 0)),
        out_shape=jax.ShapeDtypeStruct((NUM_CLASSES, 2 * DIM), jnp.float32),
    )(amp_row, freq_row)


@jax.jit
def _run(words_flat, tab):
    mesh = plsc.VectorSubcoreMesh(core_axis_name="c", subcore_axis_name="s")
    call = functools.partial(
        pl.kernel,
        mesh=mesh,
        compiler_params=pltpu.CompilerParams(use_tc_tiling_on_sc=False),
        out_type=jax.ShapeDtypeStruct((BATCH, SEQ, 2 * DIM), jnp.float32),
        scratch_types=[
            pltpu.VMEM((CHUNK,), jnp.int32),
            pltpu.VMEM((CHUNK,), jnp.int32),
            pltpu.VMEM((CHUNK, 2 * DIM), jnp.float32),
            pltpu.VMEM((CHUNK, 2 * DIM), jnp.float32),
            pltpu.VMEM((CHUNK, 2 * DIM), jnp.float32),
            pltpu.VMEM((CHUNK, 2 * DIM), jnp.float32),
            pltpu.SemaphoreType.DMA,
            pltpu.SemaphoreType.DMA,
            pltpu.SemaphoreType.DMA,
            pltpu.SemaphoreType.DMA,
        ],
    )(_body)
    return call(words_flat, tab)


def kernel(words, amp_table, freq_table):
    words_flat = words.T.reshape(BL).astype(jnp.int32)
    tab = _build_table(amp_table, freq_table)
    return _run(words_flat, tab)


# speedup vs baseline: 2.0594x; 2.0594x over previous
import functools

import jax
import jax.numpy as jnp
from jax import lax
from jax.experimental import pallas as pl
from jax.experimental.pallas import tpu as pltpu
from jax.experimental.pallas import tpu_sc as plsc

NUM_CLASSES = 1000000
DIM = 64
BATCH = 1024
SEQ = 200
BL = BATCH * SEQ

NUM_WORKERS = 32
ROWS_PER_WORKER = BL // NUM_WORKERS
CHUNK = 128
NUM_CHUNKS = ROWS_PER_WORKER // CHUNK
NUM_PAIRS = NUM_CHUNKS // 2

_INV2PI = 0.15915494309189535
_MAGIC = 1.5 * 2.0**23
_C1 = 6.28125
_C2 = 6.283185307179586 - 6.28125

_S0, _S1, _S2, _S3 = (9.9945015e-01, -1.6583844e-01, 7.9985755e-03,
                      -1.4774044e-04)
_D0, _D1, _D2, _D3, _D4 = (9.9997109e-01, -4.9983761e-01, 4.1522305e-02,
                           -1.3441069e-03, 1.9065215e-05)


def _sincos(p):
    k = (p * _INV2PI + _MAGIC) - _MAGIC
    r = (p - k * _C1) - k * _C2
    t = r * r
    t2 = t * t
    s = ((_S0 + _S1 * t) + t2 * (_S2 + _S3 * t)) * r
    c_ = (_D0 + _D1 * t) + t2 * ((_D2 + _D3 * t) + t2 * _D4)
    return s, c_


def _body(words_hbm, tab_hbm, out_hbm,
          idx0, idx1, buf0, buf1, ob0, ob1,
          gsem0, gsem1, wsem0, wsem1):
    nc = 2
    wid = lax.axis_index("s") * nc + lax.axis_index("c")
    chunk0 = wid * NUM_CHUNKS

    def compute(buf, ob, l):
        posv = jnp.full((16,), (l + 1).astype(jnp.float32), jnp.float32)

        @plsc.parallel_loop(0, CHUNK, unroll=8)
        def row_body(r):
            for j in range(DIM // 16):
                a = buf[r, pl.ds(16 * j, 16)]
                f = buf[r, pl.ds(DIM + 16 * j, 16)]
                s, c = _sincos(f * posv)
                ob[r, pl.ds(16 * j, 16)] = a * c
                ob[r, pl.ds(DIM + 16 * j, 16)] = a * s

    def start_gather(g, idx, buf, gsem):
        pltpu.sync_copy(words_hbm.at[pl.ds(g * CHUNK, CHUNK)], idx)
        pltpu.make_async_copy(tab_hbm.at[idx], buf, gsem).start()

    def wait_gather(idx, buf, gsem):
        pltpu.make_async_copy(tab_hbm.at[idx], buf, gsem).wait()

    def out_block(g):
        q0 = g * CHUNK
        l = q0 // BATCH
        b0 = q0 % BATCH
        return out_hbm.at[pl.ds(b0, CHUNK), l], l

    start_gather(chunk0, idx0, buf0, gsem0)

    def pair_body(i, carry):
        g0 = chunk0 + 2 * i
        g1 = g0 + 1
        dst0, l0 = out_block(g0)
        dst1, l1 = out_block(g1)

        start_gather(g1, idx1, buf1, gsem1)

        wait_gather(idx0, buf0, gsem0)

        @pl.when(i > 0)
        def _():
            pltpu.make_async_copy(ob0, dst0, wsem0).wait()

        compute(buf0, ob0, l0)
        pltpu.make_async_copy(ob0, dst0, wsem0).start()

        @pl.when(i < NUM_PAIRS - 1)
        def _():
            start_gather(g0 + 2, idx0, buf0, gsem0)

        wait_gather(idx1, buf1, gsem1)

        @pl.when(i > 0)
        def _():
            pltpu.make_async_copy(ob1, dst1, wsem1).wait()

        compute(buf1, ob1, l1)
        pltpu.make_async_copy(ob1, dst1, wsem1).start()
        return carry

    lax.fori_loop(0, NUM_PAIRS, pair_body, 0)

    dstl0, _ = out_block(chunk0 + NUM_CHUNKS - 2)
    dstl1, _ = out_block(chunk0 + NUM_CHUNKS - 1)
    pltpu.make_async_copy(ob0, dstl0, wsem0).wait()
    pltpu.make_async_copy(ob1, dstl1, wsem1).wait()


@jax.jit
def _run(words_flat, tab):
    mesh = plsc.VectorSubcoreMesh(core_axis_name="c", subcore_axis_name="s")
    call = functools.partial(
        pl.kernel,
        mesh=mesh,
        compiler_params=pltpu.CompilerParams(use_tc_tiling_on_sc=False),
        out_type=jax.ShapeDtypeStruct((BATCH, SEQ, 2 * DIM), jnp.float32),
        scratch_types=[
            pltpu.VMEM((CHUNK,), jnp.int32),
            pltpu.VMEM((CHUNK,), jnp.int32),
            pltpu.VMEM((CHUNK, 2 * DIM), jnp.float32),
            pltpu.VMEM((CHUNK, 2 * DIM), jnp.float32),
            pltpu.VMEM((CHUNK, 2 * DIM), jnp.float32),
            pltpu.VMEM((CHUNK, 2 * DIM), jnp.float32),
            pltpu.SemaphoreType.DMA,
            pltpu.SemaphoreType.DMA,
            pltpu.SemaphoreType.DMA,
            pltpu.SemaphoreType.DMA,
        ],
    )(_body)
    return call(words_flat, tab)


def kernel(words, amp_table, freq_table):
    words_flat = words.T.reshape(BL).astype(jnp.int32)
    tab = jnp.concatenate([amp_table.T, freq_table.T], axis=0).T
    return _run(words_flat, tab)
